# baseline (speedup 1.0000x reference)
import functools

import jax
import jax.numpy as jnp
from jax import lax
from jax.experimental import pallas as pl
from jax.experimental.pallas import tpu as pltpu
from jax.experimental.pallas import tpu_sc as plsc

VOCAB = 110000
EMB = 64
H = 128
NCLS = 20
B = 1024
L = 200

TOTAL = B * L
NW = 32
CH = 128
HL = 100
NCH = 64
ROWS_W = 32

LT = 8
NLC = L // LT
BP = B // 2
DP = 2 * EMB
HP = 2 * H


def _sc_gather_body(table_hbm, idx_hbm, out_hbm, idx_v, rows_v, sem):
    wid = lax.axis_index("s") * 2 + lax.axis_index("c")
    pltpu.sync_copy(idx_hbm.at[wid], idx_v)

    def body(m, _):
        c = pltpu.async_copy(table_hbm.at[idx_v.at[m]], rows_v, sem)
        b = wid * ROWS_W + m // 2
        t0 = (m % 2) * HL
        k = b % (B // 2)
        half = b // (B // 2)
        c.wait()
        pltpu.sync_copy(rows_v.at[pl.ds(0, HL)],
                        out_hbm.at[pl.ds(t0, HL), k, pl.ds(half * EMB, EMB)])
        return 0

    lax.fori_loop(0, NCH, body, 0)


@functools.cache
def _sc_gather():
    return pl.kernel(
        _sc_gather_body,
        out_type=jax.ShapeDtypeStruct((L, B // 2, DP), jnp.float32),
        mesh=plsc.VectorSubcoreMesh(core_axis_name="c", subcore_axis_name="s"),
        scratch_types=[
            pltpu.VMEM((NCH, CH), jnp.int32),
            pltpu.VMEM((CH, EMB), jnp.float32),
            pltpu.SemaphoreType.DMA,
        ],
        compiler_params=pltpu.CompilerParams(use_tc_tiling_on_sc=False),
    )


def _rnn_body(e_ref, w1_ref, wh1_ref, w2_ref, wh2_ref, fct_ref,
              b1_ref, b2_ref, fcb_ref, out_ref, h1_ref, h2_ref, h1buf_ref):
    lc = pl.program_id(0)

    @pl.when(lc == 0)
    def _():
        h1_ref[...] = jnp.zeros_like(h1_ref)
        h2_ref[...] = jnp.zeros_like(h2_ref)

    e = e_ref[...].reshape(LT * BP, DP)
    xp1 = jnp.dot(e, w1_ref[...], preferred_element_type=jnp.float32)
    xp1 = xp1 + b1_ref[...]

    h1 = h1_ref[...]
    for t in range(LT):
        h1 = jnp.tanh(
            xp1[t * BP:(t + 1) * BP]
            + jnp.dot(h1, wh1_ref[...], preferred_element_type=jnp.float32))
        h1buf_ref[t * BP:(t + 1) * BP] = h1
    h1_ref[...] = h1

    xp2 = jnp.dot(h1buf_ref[...], w2_ref[...],
                  preferred_element_type=jnp.float32)
    xp2 = xp2 + b2_ref[...]

    h2 = h2_ref[...]
    for t in range(LT):
        h2 = jnp.tanh(
            xp2[t * BP:(t + 1) * BP]
            + jnp.dot(h2, wh2_ref[...], preferred_element_type=jnp.float32))
    h2_ref[...] = h2

    @pl.when(lc == NLC - 1)
    def _():
        out_ref[...] = (
            jnp.dot(h2, fct_ref[...], preferred_element_type=jnp.float32)
            + fcb_ref[...])


_rnn_call = pl.pallas_call(
    _rnn_body,
    grid=(NLC,),
    in_specs=[
        pl.BlockSpec((LT, BP, DP), lambda l: (l, 0, 0)),
        pl.BlockSpec((DP, HP), lambda l: (0, 0)),
        pl.BlockSpec((HP, HP), lambda l: (0, 0)),
        pl.BlockSpec((HP, HP), lambda l: (0, 0)),
        pl.BlockSpec((HP, HP), lambda l: (0, 0)),
        pl.BlockSpec((HP, 2 * NCLS), lambda l: (0, 0)),
        pl.BlockSpec((1, HP), lambda l: (0, 0)),
        pl.BlockSpec((1, HP), lambda l: (0, 0)),
        pl.BlockSpec((1, 2 * NCLS), lambda l: (0, 0)),
    ],
    out_specs=pl.BlockSpec((BP, 2 * NCLS), lambda l: (0, 0)),
    out_shape=jax.ShapeDtypeStruct((BP, 2 * NCLS), jnp.float32),
    scratch_shapes=[
        pltpu.VMEM((BP, HP), jnp.float32),
        pltpu.VMEM((BP, HP), jnp.float32),
        pltpu.VMEM((LT * BP, HP), jnp.float32),
    ],
    compiler_params=pltpu.CompilerParams(
        dimension_semantics=("arbitrary",)),
)


def _blkdiag(a):
    m, n = a.shape
    z = jnp.zeros((m, n), a.dtype)
    return jnp.concatenate(
        [jnp.concatenate([a, z], axis=1), jnp.concatenate([z, a], axis=1)],
        axis=0)


def kernel(x, emb, w_ih1, w_hh1, b_ih1, b_hh1,
           w_ih2, w_hh2, b_ih2, b_hh2, fc_w, fc_b):
    idxp = (jnp.pad(x.astype(jnp.int32).reshape(B, 2, HL),
                    ((0, 0), (0, 0), (0, CH - HL)), mode="edge")
            .reshape(NW, NCH, CH))
    e = _sc_gather()(emb, idxp)

    b1 = jnp.concatenate([b_ih1 + b_hh1] * 2)[None, :]
    b2 = jnp.concatenate([b_ih2 + b_hh2] * 2)[None, :]
    fcb = jnp.concatenate([fc_b] * 2)[None, :]
    out = _rnn_call(
        e,
        _blkdiag(w_ih1.T), _blkdiag(w_hh1.T),
        _blkdiag(w_ih2.T), _blkdiag(w_hh2.T), _blkdiag(fc_w.T),
        b1, b2, fcb)
    return out.reshape(BP, 2, NCLS).transpose(1, 0, 2).reshape(B, NCLS)


# speedup vs baseline: 1.3808x; 1.3808x over previous
import functools

import jax
import jax.numpy as jnp
from jax import lax
from jax.experimental import pallas as pl
from jax.experimental.pallas import tpu as pltpu
from jax.experimental.pallas import tpu_sc as plsc

VOCAB = 110000
EMB = 64
H = 128
NCLS = 20
B = 1024
L = 200

TOTAL = B * L
NW = 32
PER_W = TOTAL // NW
CH = 128
NCH = PER_W // CH
NCHP = 56

BM = 5000
NMC = VOCAB // BM

LT = 8
NLC = L // LT


def _proj_body(e_ref, w_ref, b_ref, p_ref):
    p_ref[...] = (
        jnp.dot(e_ref[...], w_ref[...], preferred_element_type=jnp.float32)
        + b_ref[...])


_proj_call = pl.pallas_call(
    _proj_body,
    grid=(NMC,),
    in_specs=[
        pl.BlockSpec((BM, EMB), lambda i: (i, 0)),
        pl.BlockSpec((EMB, H), lambda i: (0, 0)),
        pl.BlockSpec((1, H), lambda i: (0, 0)),
    ],
    out_specs=pl.BlockSpec((BM, H), lambda i: (i, 0)),
    out_shape=jax.ShapeDtypeStruct((VOCAB, H), jnp.float32),
)


def _sc_gather_body(table_hbm, idx_hbm, out_hbm,
                    idx_v, rows_0, rows_1, sem_0, sem_1):
    wid = lax.axis_index("s") * 2 + lax.axis_index("c")
    pltpu.sync_copy(idx_hbm.at[wid], idx_v)
    base = wid * NCH

    def fire(j, buf, sem):
        jc = jnp.minimum(j, NCH - 1)
        return pltpu.async_copy(table_hbm.at[idx_v.at[jc]], buf, sem)

    def drain_scatter(j, buf, sem):
        pltpu.make_async_copy(table_hbm.at[idx_v.at[0]], buf, sem).wait()
        pltpu.sync_copy(buf, out_hbm.at[pl.ds((base + j) * CH, CH)])

    fire(0, rows_0, sem_0)

    def body(i, _):
        a = 2 * i
        fire(a + 1, rows_1, sem_1)
        drain_scatter(a, rows_0, sem_0)
        fire(a + 2, rows_0, sem_0)
        drain_scatter(a + 1, rows_1, sem_1)
        return 0

    lax.fori_loop(0, NCH // 2, body, 0)
    pltpu.make_async_copy(table_hbm.at[idx_v.at[0]], rows_0, sem_0).wait()


@functools.cache
def _sc_gather():
    return pl.kernel(
        _sc_gather_body,
        out_type=jax.ShapeDtypeStruct((TOTAL, H), jnp.float32),
        mesh=plsc.VectorSubcoreMesh(core_axis_name="c", subcore_axis_name="s"),
        scratch_types=[
            pltpu.VMEM((NCHP, CH), jnp.int32),
            pltpu.VMEM((CH, H), jnp.float32),
            pltpu.VMEM((CH, H), jnp.float32),
            pltpu.SemaphoreType.DMA,
            pltpu.SemaphoreType.DMA,
        ],
        compiler_params=pltpu.CompilerParams(use_tc_tiling_on_sc=False),
    )


def _rnn_body(xp1_ref, wh1_ref, w2_ref, wh2_ref, fct_ref,
              b2_ref, fcb_ref, out_ref, h1_ref, h2_ref):
    lc = pl.program_id(0)

    @pl.when(lc == 0)
    def _():
        h1_ref[...] = jnp.zeros_like(h1_ref)
        h2_ref[...] = jnp.zeros_like(h2_ref)

    h1 = h1_ref[...]
    h2 = h2_ref[...]
    b2 = b2_ref[...]
    for t in range(LT):
        h1 = jnp.tanh(
            xp1_ref[t * B:(t + 1) * B]
            + jnp.dot(h1, wh1_ref[...], preferred_element_type=jnp.float32))
        h2 = jnp.tanh(
            jnp.dot(h1, w2_ref[...], preferred_element_type=jnp.float32)
            + b2
            + jnp.dot(h2, wh2_ref[...], preferred_element_type=jnp.float32))
    h1_ref[...] = h1
    h2_ref[...] = h2

    @pl.when(lc == NLC - 1)
    def _():
        out_ref[...] = (
            jnp.dot(h2, fct_ref[...], preferred_element_type=jnp.float32)
            + fcb_ref[...])


_rnn_call = pl.pallas_call(
    _rnn_body,
    grid=(NLC,),
    in_specs=[
        pl.BlockSpec((LT * B, H), lambda l: (l, 0)),
        pl.BlockSpec((H, H), lambda l: (0, 0)),
        pl.BlockSpec((H, H), lambda l: (0, 0)),
        pl.BlockSpec((H, H), lambda l: (0, 0)),
        pl.BlockSpec((H, NCLS), lambda l: (0, 0)),
        pl.BlockSpec((1, H), lambda l: (0, 0)),
        pl.BlockSpec((1, NCLS), lambda l: (0, 0)),
    ],
    out_specs=pl.BlockSpec((B, NCLS), lambda l: (0, 0)),
    out_shape=jax.ShapeDtypeStruct((B, NCLS), jnp.float32),
    scratch_shapes=[
        pltpu.VMEM((B, H), jnp.float32),
        pltpu.VMEM((B, H), jnp.float32),
    ],
    compiler_params=pltpu.CompilerParams(
        dimension_semantics=("arbitrary",)),
)


def kernel(x, emb, w_ih1, w_hh1, b_ih1, b_hh1,
           w_ih2, w_hh2, b_ih2, b_hh2, fc_w, fc_b):
    p = _proj_call(emb, w_ih1.T, (b_ih1 + b_hh1)[None, :])

    idx3d = jnp.pad(x.T.reshape(NW, NCH, CH).astype(jnp.int32),
                    ((0, 0), (0, NCHP - NCH), (0, 0)))
    xp1 = _sc_gather()(p, idx3d)

    out = _rnn_call(
        xp1,
        w_hh1.T, w_ih2.T, w_hh2.T, fc_w.T,
        (b_ih2 + b_hh2)[None, :], fc_b[None, :])
    return out
